# Initial kernel scaffold; baseline (speedup 1.0000x reference)
import jax
import jax.numpy as jnp
from jax.experimental import pallas as pl


def kernel(x, W1, b1, W2, b2, Wg, bg):
    raise NotImplementedError("write your pallas kernel here")


# speedup vs baseline: 1.3652x; 1.3652x over previous
import functools

import jax
import jax.numpy as jnp
from jax.experimental import pallas as pl
from jax.experimental.pallas import tpu as pltpu

D_MODEL = 1024
D_FF = 2048
NUM_EXPERTS = 8
TOP_K = 2
TOKEN_BLOCK = 1024


def _moe_body(x_ref, wg_ref, bg_ref, w1_ref, b1_ref, w2_ref, b2_ref,
              out_ref, wts_ref):
    e = pl.program_id(1)
    cols8 = jax.lax.broadcasted_iota(jnp.int32, (TOKEN_BLOCK, NUM_EXPERTS), 1)

    @pl.when(e == 0)
    def _gate():
        xb = x_ref[...].astype(jnp.bfloat16)
        scores = jnp.dot(xb, wg_ref[...].astype(jnp.bfloat16),
                         preferred_element_type=jnp.float32) + bg_ref[...]
        m = jnp.max(scores, axis=1, keepdims=True)
        p = jnp.exp(scores - m)
        p = p / jnp.sum(p, axis=1, keepdims=True)
        v0 = jnp.max(p, axis=1, keepdims=True)
        a0 = jnp.min(jnp.where(p >= v0, cols8, NUM_EXPERTS),
                     axis=1, keepdims=True)
        p1 = jnp.where(cols8 == a0, -jnp.inf, p)
        v1 = jnp.max(p1, axis=1, keepdims=True)
        a1 = jnp.min(jnp.where(p1 >= v1, cols8, NUM_EXPERTS),
                     axis=1, keepdims=True)
        wts_ref[...] = (jnp.where(cols8 == a0, v0, 0.0)
                        + jnp.where(cols8 == a1, v1, 0.0))

    w = jnp.sum(wts_ref[...] * (cols8 == e).astype(jnp.float32),
                axis=1, keepdims=True)
    xb16 = x_ref[...].astype(jnp.bfloat16)
    h = jnp.dot(xb16, w1_ref[0], preferred_element_type=jnp.float32)
    h = jnp.maximum(h + b1_ref[0], 0.0)
    y = jnp.dot(h.astype(jnp.bfloat16), w2_ref[0],
                preferred_element_type=jnp.float32) + b2_ref[0]
    contrib = y * w

    @pl.when(e == 0)
    def _init():
        out_ref[...] = contrib

    @pl.when(e > 0)
    def _acc():
        out_ref[...] += contrib


@jax.jit
def kernel(x, W1, b1, W2, b2, Wg, bg):
    B, N, D = x.shape
    tokens = B * N
    x2 = x.reshape(tokens, D)
    nb = tokens // TOKEN_BLOCK
    grid = (nb, NUM_EXPERTS)
    out = pl.pallas_call(
        _moe_body,
        grid=grid,
        in_specs=[
            pl.BlockSpec((TOKEN_BLOCK, D_MODEL), lambda i, e: (i, 0)),
            pl.BlockSpec((D_MODEL, NUM_EXPERTS), lambda i, e: (0, 0)),
            pl.BlockSpec((1, NUM_EXPERTS), lambda i, e: (0, 0)),
            pl.BlockSpec((1, D_MODEL, D_FF), lambda i, e: (e, 0, 0)),
            pl.BlockSpec((1, 1, D_FF), lambda i, e: (e, 0, 0)),
            pl.BlockSpec((1, D_FF, D_MODEL), lambda i, e: (e, 0, 0)),
            pl.BlockSpec((1, 1, D_MODEL), lambda i, e: (e, 0, 0)),
        ],
        out_specs=pl.BlockSpec((TOKEN_BLOCK, D_MODEL), lambda i, e: (i, 0)),
        out_shape=jax.ShapeDtypeStruct((tokens, D_MODEL), jnp.float32),
        scratch_shapes=[pltpu.VMEM((TOKEN_BLOCK, NUM_EXPERTS), jnp.float32)],
        compiler_params=pltpu.CompilerParams(
            dimension_semantics=("arbitrary", "arbitrary"),
        ),
    )(x2, Wg, bg.reshape(1, NUM_EXPERTS),
      W1.astype(jnp.bfloat16), b1.reshape(NUM_EXPERTS, 1, D_FF),
      W2.astype(jnp.bfloat16), b2.reshape(NUM_EXPERTS, 1, D_MODEL))
    return out.reshape(B, N, D)


# speedup vs baseline: 1.7438x; 1.2773x over previous
import functools

import jax
import jax.numpy as jnp
from jax import lax
from jax.experimental import pallas as pl
from jax.experimental.pallas import tpu as pltpu
from jax.experimental.pallas import tpu_sc as plsc

D_MODEL = 1024
D_FF = 2048
NUM_EXPERTS = 8
TOP_K = 2
TOKENS = 4096
ROW_TILE = 256
NUM_TILES = 40
RSORT = NUM_TILES * ROW_TILE
NW = 32
TPW = TOKENS // NW


def _route_body(x_ref, wg_ref, bg_ref, pos_ref, sb_ref, eot_ref):
    xb = x_ref[...].astype(jnp.bfloat16)
    scores = jnp.dot(xb, wg_ref[...].astype(jnp.bfloat16),
                     preferred_element_type=jnp.float32) + bg_ref[...]
    cols = lax.broadcasted_iota(jnp.int32, (TOKENS, NUM_EXPERTS), 1)
    m = jnp.max(scores, axis=1, keepdims=True)
    p = jnp.exp(scores - m)
    p = p / jnp.sum(p, axis=1, keepdims=True)
    v0 = jnp.max(p, axis=1, keepdims=True)
    a0 = jnp.min(jnp.where(p >= v0, cols, NUM_EXPERTS), axis=1, keepdims=True)
    p1m = jnp.where(cols == a0, -jnp.inf, p)
    v1 = jnp.max(p1m, axis=1, keepdims=True)
    a1 = jnp.min(jnp.where(p1m >= v1, cols, NUM_EXPERTS), axis=1, keepdims=True)
    oh0 = (cols == a0).astype(jnp.float32)
    oh1 = (cols == a1).astype(jnp.float32)
    sel = oh0 + oh1
    c = sel
    d = 1
    while d < TOKENS:
        z = jnp.zeros((d, NUM_EXPERTS), jnp.float32)
        c = c + jnp.concatenate([z, c[:TOKENS - d, :]], axis=0)
        d *= 2
    rank = c - sel
    counts = jnp.sum(sel, axis=0, keepdims=True)
    cp = jnp.floor((counts + (ROW_TILE - 1)) / ROW_TILE) * ROW_TILE
    rr = lax.broadcasted_iota(jnp.int32, (NUM_EXPERTS, NUM_EXPERTS), 0)
    cc = lax.broadcasted_iota(jnp.int32, (NUM_EXPERTS, NUM_EXPERTS), 1)
    tri = (rr < cc).astype(jnp.float32)
    op = jnp.dot(cp, tri, preferred_element_type=jnp.float32)
    opend = op + cp
    ident = (rr == cc).astype(jnp.float32)
    opend_t = lax.dot_general(ident, opend, (((1,), (1,)), ((), ())),
                              preferred_element_type=jnp.float32)
    tile_start = lax.broadcasted_iota(
        jnp.int32, (NUM_EXPERTS, 64), 1).astype(jnp.float32) * ROW_TILE
    eot = jnp.sum((tile_start >= opend_t).astype(jnp.float32),
                  axis=0, keepdims=True)
    eot_ref[...] = jnp.minimum(eot, NUM_EXPERTS - 1.0).astype(jnp.int32)
    posf = op + rank
    pw0 = jnp.sum(posf * oh0, axis=1, keepdims=True)
    pw1 = jnp.sum(posf * oh1, axis=1, keepdims=True)
    pos_ref[...] = jnp.concatenate([pw0, pw1], axis=1).astype(jnp.int32)
    s0 = jnp.sum(p * oh0, axis=1, keepdims=True)
    s1 = jnp.sum(p * oh1, axis=1, keepdims=True)
    sb_ref[...] = jnp.concatenate(
        [jnp.broadcast_to(s0, (TOKENS, 16)),
         jnp.broadcast_to(s1, (TOKENS, 16))], axis=0)


def _route(x2, Wg, bg):
    return pl.pallas_call(
        _route_body,
        out_shape=[
            jax.ShapeDtypeStruct((TOKENS, 2), jnp.int32),
            jax.ShapeDtypeStruct((2 * TOKENS, 16), jnp.float32),
            jax.ShapeDtypeStruct((1, 64), jnp.int32),
        ],
    )(x2, Wg, bg.reshape(1, NUM_EXPERTS))


_DCH = 64


def _dispatch_body(x_hbm, pos_hbm, xs_hbm, xbuf, idx0, idx1, s0, s1):
    wid = lax.axis_index("s") * 2 + lax.axis_index("c")
    for ci in range(TPW // _DCH):
        base = wid * TPW + ci * _DCH
        pltpu.sync_copy(pos_hbm.at[0, pl.ds(base, _DCH)], idx0)
        pltpu.sync_copy(pos_hbm.at[1, pl.ds(base, _DCH)], idx1)
        pltpu.sync_copy(x_hbm.at[pl.ds(base, _DCH)], xbuf)
        c0 = pltpu.async_copy(xbuf, xs_hbm.at[idx0], s0)
        c1 = pltpu.async_copy(xbuf, xs_hbm.at[idx1], s1)
        c0.wait()
        c1.wait()


def _dispatch(x2, pos_t):
    mesh = plsc.VectorSubcoreMesh(core_axis_name="c", subcore_axis_name="s")
    return pl.kernel(
        _dispatch_body,
        out_type=jax.ShapeDtypeStruct((RSORT, D_MODEL), jnp.float32),
        mesh=mesh,
        scratch_types=[
            pltpu.VMEM((_DCH, D_MODEL), jnp.float32),
            pltpu.VMEM((_DCH,), jnp.int32),
            pltpu.VMEM((_DCH,), jnp.int32),
            pltpu.SemaphoreType.DMA,
            pltpu.SemaphoreType.DMA,
        ],
    )(x2, pos_t)


def _ffn_body(eot_ref, xs_ref, w1_ref, b1_ref, w2_ref, b2_ref, out_ref):
    del eot_ref
    xb = xs_ref[...].astype(jnp.bfloat16)
    h = jnp.dot(xb, w1_ref[0], preferred_element_type=jnp.float32)
    h = jnp.maximum(h + b1_ref[0], 0.0)
    out_ref[...] = jnp.dot(h.astype(jnp.bfloat16), w2_ref[0],
                           preferred_element_type=jnp.float32) + b2_ref[0]


def _ffn(eot, xs, W1b, b1r, W2b, b2r):
    grid_spec = pltpu.PrefetchScalarGridSpec(
        num_scalar_prefetch=1,
        grid=(NUM_TILES,),
        in_specs=[
            pl.BlockSpec((ROW_TILE, D_MODEL), lambda i, eot: (i, 0)),
            pl.BlockSpec((1, D_MODEL, D_FF), lambda i, eot: (eot[i], 0, 0)),
            pl.BlockSpec((1, 1, D_FF), lambda i, eot: (eot[i], 0, 0)),
            pl.BlockSpec((1, D_FF, D_MODEL), lambda i, eot: (eot[i], 0, 0)),
            pl.BlockSpec((1, 1, D_MODEL), lambda i, eot: (eot[i], 0, 0)),
        ],
        out_specs=pl.BlockSpec((ROW_TILE, D_MODEL), lambda i, eot: (i, 0)),
    )
    return pl.pallas_call(
        _ffn_body,
        grid_spec=grid_spec,
        out_shape=jax.ShapeDtypeStruct((RSORT, D_MODEL), jnp.float32),
        compiler_params=pltpu.CompilerParams(
            dimension_semantics=("arbitrary",),
        ),
    )(eot, xs, W1b, b1r, W2b, b2r)


_CCH = 16


def _combine_body(ys_hbm, pos_hbm, sb_hbm, out_hbm,
                  y0buf, y1buf, sbuf0, sbuf1, idx0, idx1, s0, s1):
    wid = lax.axis_index("s") * 2 + lax.axis_index("c")
    for ci in range(TPW // _CCH):
        base = wid * TPW + ci * _CCH
        pltpu.sync_copy(pos_hbm.at[0, pl.ds(base, _CCH)], idx0)
        pltpu.sync_copy(pos_hbm.at[1, pl.ds(base, _CCH)], idx1)
        pltpu.sync_copy(sb_hbm.at[pl.ds(base, _CCH)], sbuf0)
        pltpu.sync_copy(sb_hbm.at[pl.ds(TOKENS + base, _CCH)], sbuf1)
        g0 = pltpu.async_copy(ys_hbm.at[idx0], y0buf, s0)
        g1 = pltpu.async_copy(ys_hbm.at[idx1], y1buf, s1)
        g0.wait()
        g1.wait()

        def row_body(j, carry):
            sv0 = sbuf0[j, pl.ds(0, 16)]
            sv1 = sbuf1[j, pl.ds(0, 16)]
            for cc in range(D_MODEL // 16):
                sl = pl.ds(cc * 16, 16)
                y0buf[j, sl] = y0buf[j, sl] * sv0 + y1buf[j, sl] * sv1
            return carry

        lax.fori_loop(0, _CCH, row_body, 0)
        pltpu.sync_copy(y0buf, out_hbm.at[pl.ds(base, _CCH)])


def _combine(ys, pos_t, sb):
    mesh = plsc.VectorSubcoreMesh(core_axis_name="c", subcore_axis_name="s")
    return pl.kernel(
        _combine_body,
        out_type=jax.ShapeDtypeStruct((TOKENS, D_MODEL), jnp.float32),
        mesh=mesh,
        scratch_types=[
            pltpu.VMEM((_CCH, D_MODEL), jnp.float32),
            pltpu.VMEM((_CCH, D_MODEL), jnp.float32),
            pltpu.VMEM((_CCH, 16), jnp.float32),
            pltpu.VMEM((_CCH, 16), jnp.float32),
            pltpu.VMEM((_CCH,), jnp.int32),
            pltpu.VMEM((_CCH,), jnp.int32),
            pltpu.SemaphoreType.DMA,
            pltpu.SemaphoreType.DMA,
        ],
    )(ys, pos_t, sb)


@jax.jit
def kernel(x, W1, b1, W2, b2, Wg, bg):
    B, N, D = x.shape
    x2 = x.reshape(B * N, D)
    pos01, sb, eot64 = _route(x2, Wg, bg)
    pos_t = pos01.T
    eot = eot64.reshape(64)[:NUM_TILES]
    xs = _dispatch(x2, pos_t)
    ys = _ffn(eot, xs, W1.astype(jnp.bfloat16), b1.reshape(NUM_EXPERTS, 1, D_FF),
              W2.astype(jnp.bfloat16), b2.reshape(NUM_EXPERTS, 1, D_MODEL))
    out2 = _combine(ys, pos_t, sb)
    return out2.reshape(B, N, D)


# speedup vs baseline: 1.9331x; 1.1086x over previous
import functools

import jax
import jax.numpy as jnp
from jax import lax
from jax.experimental import pallas as pl
from jax.experimental.pallas import tpu as pltpu
from jax.experimental.pallas import tpu_sc as plsc

D_MODEL = 1024
D_FF = 2048
NUM_EXPERTS = 8
TOP_K = 2
TOKENS = 4096
ROW_TILE = 256
NUM_TILES = 40
RSORT = NUM_TILES * ROW_TILE
NW = 32
TPW = TOKENS // NW


def _route_body(x_ref, wg_ref, bg_ref, pos_ref, sb_ref, eot_ref):
    xb = x_ref[...].astype(jnp.bfloat16)
    scores = jnp.dot(xb, wg_ref[...].astype(jnp.bfloat16),
                     preferred_element_type=jnp.float32) + bg_ref[...]
    cols = lax.broadcasted_iota(jnp.int32, (TOKENS, NUM_EXPERTS), 1)
    m = jnp.max(scores, axis=1, keepdims=True)
    p = jnp.exp(scores - m)
    p = p / jnp.sum(p, axis=1, keepdims=True)
    v0 = jnp.max(p, axis=1, keepdims=True)
    a0 = jnp.min(jnp.where(p >= v0, cols, NUM_EXPERTS), axis=1, keepdims=True)
    p1m = jnp.where(cols == a0, -jnp.inf, p)
    v1 = jnp.max(p1m, axis=1, keepdims=True)
    a1 = jnp.min(jnp.where(p1m >= v1, cols, NUM_EXPERTS), axis=1, keepdims=True)
    oh0 = (cols == a0).astype(jnp.float32)
    oh1 = (cols == a1).astype(jnp.float32)
    sel = oh0 + oh1
    c = sel
    d = 1
    while d < TOKENS:
        z = jnp.zeros((d, NUM_EXPERTS), jnp.float32)
        c = c + jnp.concatenate([z, c[:TOKENS - d, :]], axis=0)
        d *= 2
    rank = c - sel
    counts = jnp.sum(sel, axis=0, keepdims=True)
    cp = jnp.floor((counts + (ROW_TILE - 1)) / ROW_TILE) * ROW_TILE
    rr = lax.broadcasted_iota(jnp.int32, (NUM_EXPERTS, NUM_EXPERTS), 0)
    cc = lax.broadcasted_iota(jnp.int32, (NUM_EXPERTS, NUM_EXPERTS), 1)
    tri = (rr < cc).astype(jnp.float32)
    op = jnp.dot(cp, tri, preferred_element_type=jnp.float32)
    opend = op + cp
    ident = (rr == cc).astype(jnp.float32)
    opend_t = lax.dot_general(ident, opend, (((1,), (1,)), ((), ())),
                              preferred_element_type=jnp.float32)
    tile_start = lax.broadcasted_iota(
        jnp.int32, (NUM_EXPERTS, 64), 1).astype(jnp.float32) * ROW_TILE
    eot = jnp.sum((tile_start >= opend_t).astype(jnp.float32),
                  axis=0, keepdims=True)
    eot_ref[...] = jnp.minimum(eot, NUM_EXPERTS - 1.0).astype(jnp.int32)
    posf = op + rank
    pw0 = jnp.sum(posf * oh0, axis=1, keepdims=True)
    pw1 = jnp.sum(posf * oh1, axis=1, keepdims=True)
    pos_ref[...] = jnp.concatenate([pw0, pw1], axis=1).astype(jnp.int32)
    s0 = jnp.sum(p * oh0, axis=1, keepdims=True)
    s1 = jnp.sum(p * oh1, axis=1, keepdims=True)
    sb_ref[...] = jnp.concatenate(
        [jnp.broadcast_to(s0, (TOKENS, 16)),
         jnp.broadcast_to(s1, (TOKENS, 16))], axis=0)


def _route(x2, Wg, bg):
    return pl.pallas_call(
        _route_body,
        out_shape=[
            jax.ShapeDtypeStruct((TOKENS, 2), jnp.int32),
            jax.ShapeDtypeStruct((2 * TOKENS, 16), jnp.float32),
            jax.ShapeDtypeStruct((1, 64), jnp.int32),
        ],
    )(x2, Wg, bg.reshape(1, NUM_EXPERTS))


_DCH = 32
_DNC = TPW // _DCH


def _dispatch_body(x_hbm, pos_hbm, xs_hbm, xb0, xb1, *rest):
    idx = rest[:2 * _DNC]
    isem, lsem0, lsem1, ssem0, ssem1 = rest[2 * _DNC:]
    xbufs = (xb0, xb1)
    lsems = (lsem0, lsem1)
    ssems = (ssem0, ssem1)
    wid = lax.axis_index("s") * 2 + lax.axis_index("c")
    base0 = wid * TPW
    icp = []
    for ci in range(_DNC):
        for k in range(2):
            icp.append(pltpu.async_copy(
                pos_hbm.at[k, pl.ds(base0 + ci * _DCH, _DCH)],
                idx[k * _DNC + ci], isem))
    loads = [None, None]
    scats = [None, None, None, None]
    for ci in range(2):
        loads[ci] = pltpu.async_copy(
            x_hbm.at[pl.ds(base0 + ci * _DCH, _DCH)], xbufs[ci], lsems[ci])
    for c in icp:
        c.wait()
    for ci in range(_DNC):
        b = ci % 2
        loads[b].wait()
        scats[2 * b] = pltpu.async_copy(
            xbufs[b], xs_hbm.at[idx[ci]], ssems[b])
        scats[2 * b + 1] = pltpu.async_copy(
            xbufs[b], xs_hbm.at[idx[_DNC + ci]], ssems[b])
        if ci + 2 < _DNC:
            scats[2 * b].wait()
            scats[2 * b + 1].wait()
            loads[b] = pltpu.async_copy(
                x_hbm.at[pl.ds(base0 + (ci + 2) * _DCH, _DCH)],
                xbufs[b], lsems[b])
    for b in range(2):
        scats[2 * b].wait()
        scats[2 * b + 1].wait()


def _dispatch(x2, pos_t):
    mesh = plsc.VectorSubcoreMesh(core_axis_name="c", subcore_axis_name="s")
    return pl.kernel(
        _dispatch_body,
        out_type=jax.ShapeDtypeStruct((RSORT, D_MODEL), jnp.float32),
        mesh=mesh,
        scratch_types=(
            [pltpu.VMEM((_DCH, D_MODEL), jnp.float32)] * 2
            + [pltpu.VMEM((_DCH,), jnp.int32)] * (2 * _DNC)
            + [pltpu.SemaphoreType.DMA] * 5
        ),
    )(x2, pos_t)


def _ffn_body(eot_ref, xs_ref, w1_ref, b1_ref, w2_ref, b2_ref, out_ref):
    del eot_ref
    xb = xs_ref[...].astype(jnp.bfloat16)
    h = jnp.dot(xb, w1_ref[0], preferred_element_type=jnp.float32)
    h = jnp.maximum(h + b1_ref[0], 0.0)
    out_ref[...] = jnp.dot(h.astype(jnp.bfloat16), w2_ref[0],
                           preferred_element_type=jnp.float32) + b2_ref[0]


def _ffn(eot, xs, W1b, b1r, W2b, b2r):
    grid_spec = pltpu.PrefetchScalarGridSpec(
        num_scalar_prefetch=1,
        grid=(NUM_TILES,),
        in_specs=[
            pl.BlockSpec((ROW_TILE, D_MODEL), lambda i, eot: (i, 0)),
            pl.BlockSpec((1, D_MODEL, D_FF), lambda i, eot: (eot[i], 0, 0)),
            pl.BlockSpec((1, 1, D_FF), lambda i, eot: (eot[i], 0, 0)),
            pl.BlockSpec((1, D_FF, D_MODEL), lambda i, eot: (eot[i], 0, 0)),
            pl.BlockSpec((1, 1, D_MODEL), lambda i, eot: (eot[i], 0, 0)),
        ],
        out_specs=pl.BlockSpec((ROW_TILE, D_MODEL), lambda i, eot: (i, 0)),
    )
    return pl.pallas_call(
        _ffn_body,
        grid_spec=grid_spec,
        out_shape=jax.ShapeDtypeStruct((RSORT, D_MODEL), jnp.float32),
        compiler_params=pltpu.CompilerParams(
            dimension_semantics=("arbitrary",),
        ),
    )(eot, xs, W1b, b1r, W2b, b2r)


_CCH = 16


_CNC = TPW // _CCH


def _combine_body(ys_hbm, pos_hbm, sb_hbm, out_hbm,
                  y0a, y1a, y0b, y1b,
                  sbuf0, sbuf1, idx0, idx1,
                  isem, gsa, gsb, osa, osb):
    wid = lax.axis_index("s") * 2 + lax.axis_index("c")
    base0 = wid * TPW
    i0 = pltpu.async_copy(pos_hbm.at[0, pl.ds(base0, TPW)], idx0, isem)
    i1 = pltpu.async_copy(pos_hbm.at[1, pl.ds(base0, TPW)], idx1, isem)
    i2 = pltpu.async_copy(sb_hbm.at[pl.ds(base0, TPW)], sbuf0, isem)
    i3 = pltpu.async_copy(sb_hbm.at[pl.ds(TOKENS + base0, TPW)], sbuf1, isem)
    i0.wait(); i1.wait(); i2.wait(); i3.wait()
    ybufs = ((y0a, y1a), (y0b, y1b))
    gsems = (gsa, gsb)
    osems = (osa, osb)
    gaths = [None, None]
    outs = [None, None]

    def gather(ci, b):
        sl = pl.ds(ci * _CCH, _CCH)
        g0 = pltpu.async_copy(ys_hbm.at[idx0.at[sl]], ybufs[b][0], gsems[b])
        g1 = pltpu.async_copy(ys_hbm.at[idx1.at[sl]], ybufs[b][1], gsems[b])
        return (g0, g1)

    gaths[0] = gather(0, 0)
    gaths[1] = gather(1, 1)
    for ci in range(_CNC):
        b = ci % 2
        gaths[b][0].wait()
        gaths[b][1].wait()
        y0, y1 = ybufs[b]

        def row_body(j, carry, y0=y0, y1=y1, ci=ci):
            sv0 = sbuf0[pl.ds(ci * _CCH + j, 1), pl.ds(0, 16)]
            sv1 = sbuf1[pl.ds(ci * _CCH + j, 1), pl.ds(0, 16)]
            s0v = sv0.reshape((16,))
            s1v = sv1.reshape((16,))
            for cc in range(D_MODEL // 16):
                sl2 = pl.ds(cc * 16, 16)
                y0[j, sl2] = y0[j, sl2] * s0v + y1[j, sl2] * s1v
            return carry

        lax.fori_loop(0, _CCH, row_body, 0)
        outs[b] = pltpu.async_copy(
            y0, out_hbm.at[pl.ds(base0 + ci * _CCH, _CCH)], osems[b])
        if ci + 2 < _CNC:
            outs[b].wait()
            gaths[b] = gather(ci + 2, b)
    for b in range(2):
        outs[b].wait()


def _combine(ys, pos_t, sb):
    mesh = plsc.VectorSubcoreMesh(core_axis_name="c", subcore_axis_name="s")
    return pl.kernel(
        _combine_body,
        out_type=jax.ShapeDtypeStruct((TOKENS, D_MODEL), jnp.float32),
        mesh=mesh,
        scratch_types=(
            [pltpu.VMEM((_CCH, D_MODEL), jnp.float32)] * 4
            + [pltpu.VMEM((TPW, 16), jnp.float32)] * 2
            + [pltpu.VMEM((TPW,), jnp.int32)] * 2
            + [pltpu.SemaphoreType.DMA] * 5
        ),
    )(ys, pos_t, sb)


@jax.jit
def kernel(x, W1, b1, W2, b2, Wg, bg):
    B, N, D = x.shape
    x2 = x.reshape(B * N, D)
    pos01, sb, eot64 = _route(x2, Wg, bg)
    pos_t = pos01.T
    eot = eot64.reshape(64)[:NUM_TILES]
    xs = _dispatch(x2, pos_t)
    ys = _ffn(eot, xs, W1.astype(jnp.bfloat16), b1.reshape(NUM_EXPERTS, 1, D_FF),
              W2.astype(jnp.bfloat16), b2.reshape(NUM_EXPERTS, 1, D_MODEL))
    out2 = _combine(ys, pos_t, sb)
    return out2.reshape(B, N, D)


# speedup vs baseline: 2.3487x; 1.2150x over previous
import functools

import jax
import jax.numpy as jnp
from jax import lax
from jax.experimental import pallas as pl
from jax.experimental.pallas import tpu as pltpu
from jax.experimental.pallas import tpu_sc as plsc

D_MODEL = 1024
D_FF = 2048
NUM_EXPERTS = 8
TOP_K = 2
TOKENS = 4096
ROW_TILE = 256
NUM_TILES = 40
RSORT = NUM_TILES * ROW_TILE
NW = 32
TPW = TOKENS // NW


def _route_body(x_ref, wg_ref, bg_ref, pos_ref, sb_ref, eot_ref):
    xb = x_ref[...].astype(jnp.bfloat16)
    scores = jnp.dot(xb, wg_ref[...].astype(jnp.bfloat16),
                     preferred_element_type=jnp.float32) + bg_ref[...]
    cols = lax.broadcasted_iota(jnp.int32, (TOKENS, NUM_EXPERTS), 1)
    m = jnp.max(scores, axis=1, keepdims=True)
    p = jnp.exp(scores - m)
    p = p / jnp.sum(p, axis=1, keepdims=True)
    v0 = jnp.max(p, axis=1, keepdims=True)
    a0 = jnp.min(jnp.where(p >= v0, cols, NUM_EXPERTS), axis=1, keepdims=True)
    p1m = jnp.where(cols == a0, -jnp.inf, p)
    v1 = jnp.max(p1m, axis=1, keepdims=True)
    a1 = jnp.min(jnp.where(p1m >= v1, cols, NUM_EXPERTS), axis=1, keepdims=True)
    oh0 = (cols == a0).astype(jnp.float32)
    oh1 = (cols == a1).astype(jnp.float32)
    sel = oh0 + oh1
    c = sel
    d = 1
    while d < TOKENS:
        z = jnp.zeros((d, NUM_EXPERTS), jnp.float32)
        c = c + jnp.concatenate([z, c[:TOKENS - d, :]], axis=0)
        d *= 2
    rank = c - sel
    counts = jnp.sum(sel, axis=0, keepdims=True)
    cp = jnp.floor((counts + (ROW_TILE - 1)) / ROW_TILE) * ROW_TILE
    rr = lax.broadcasted_iota(jnp.int32, (NUM_EXPERTS, NUM_EXPERTS), 0)
    cc = lax.broadcasted_iota(jnp.int32, (NUM_EXPERTS, NUM_EXPERTS), 1)
    tri = (rr < cc).astype(jnp.float32)
    op = jnp.dot(cp, tri, preferred_element_type=jnp.float32)
    opend = op + cp
    ident = (rr == cc).astype(jnp.float32)
    opend_t = lax.dot_general(ident, opend, (((1,), (1,)), ((), ())),
                              preferred_element_type=jnp.float32)
    tile_start = lax.broadcasted_iota(
        jnp.int32, (NUM_EXPERTS, 64), 1).astype(jnp.float32) * ROW_TILE
    eot = jnp.sum((tile_start >= opend_t).astype(jnp.float32),
                  axis=0, keepdims=True)
    eot = jnp.minimum(eot, NUM_EXPERTS - 1.0)
    lanes64 = lax.broadcasted_iota(jnp.int32, (1, 64), 1)
    n_used = jnp.sum(cp) / ROW_TILE
    eot_ref[...] = jnp.where(lanes64 == 63, n_used, eot).astype(jnp.int32)
    posf = op + rank
    pw0 = jnp.sum(posf * oh0, axis=1, keepdims=True)
    pw1 = jnp.sum(posf * oh1, axis=1, keepdims=True)
    pos_ref[...] = jnp.concatenate([pw0, pw1], axis=1).astype(jnp.int32)
    s0 = jnp.sum(p * oh0, axis=1, keepdims=True)
    s1 = jnp.sum(p * oh1, axis=1, keepdims=True)
    sb_ref[...] = jnp.concatenate(
        [jnp.broadcast_to(s0, (TOKENS, 16)),
         jnp.broadcast_to(s1, (TOKENS, 16))], axis=0)


def _route(x2, Wg, bg):
    return pl.pallas_call(
        _route_body,
        out_shape=[
            jax.ShapeDtypeStruct((TOKENS, 2), jnp.int32),
            jax.ShapeDtypeStruct((2 * TOKENS, 16), jnp.float32),
            jax.ShapeDtypeStruct((1, 64), jnp.int32),
        ],
    )(x2, Wg, bg.reshape(1, NUM_EXPERTS))


_DCH = 32
_DNC = TPW // _DCH


def _dispatch_body(x_hbm, pos_hbm, xs_hbm, xb0, xb1, *rest):
    idx = rest[:2 * _DNC]
    isem, lsem0, lsem1, ssem0, ssem1 = rest[2 * _DNC:]
    xbufs = (xb0, xb1)
    lsems = (lsem0, lsem1)
    ssems = (ssem0, ssem1)
    wid = lax.axis_index("s") * 2 + lax.axis_index("c")
    base0 = wid * TPW
    icp = []
    for ci in range(_DNC):
        for k in range(2):
            icp.append(pltpu.async_copy(
                pos_hbm.at[k, pl.ds(base0 + ci * _DCH, _DCH)],
                idx[k * _DNC + ci], isem))
    loads = [None, None]
    scats = [None, None, None, None]
    for ci in range(2):
        loads[ci] = pltpu.async_copy(
            x_hbm.at[pl.ds(base0 + ci * _DCH, _DCH)], xbufs[ci], lsems[ci])
    for c in icp:
        c.wait()
    for ci in range(_DNC):
        b = ci % 2
        loads[b].wait()
        scats[2 * b] = pltpu.async_copy(
            xbufs[b], xs_hbm.at[idx[ci]], ssems[b])
        scats[2 * b + 1] = pltpu.async_copy(
            xbufs[b], xs_hbm.at[idx[_DNC + ci]], ssems[b])
        if ci + 2 < _DNC:
            scats[2 * b].wait()
            scats[2 * b + 1].wait()
            loads[b] = pltpu.async_copy(
                x_hbm.at[pl.ds(base0 + (ci + 2) * _DCH, _DCH)],
                xbufs[b], lsems[b])
    for b in range(2):
        scats[2 * b].wait()
        scats[2 * b + 1].wait()


def _dispatch(x2, pos_t):
    mesh = plsc.VectorSubcoreMesh(core_axis_name="c", subcore_axis_name="s")
    return pl.kernel(
        _dispatch_body,
        out_type=jax.ShapeDtypeStruct((RSORT, D_MODEL), jnp.float32),
        mesh=mesh,
        scratch_types=(
            [pltpu.VMEM((_DCH, D_MODEL), jnp.float32)] * 2
            + [pltpu.VMEM((_DCH,), jnp.int32)] * (2 * _DNC)
            + [pltpu.SemaphoreType.DMA] * 5
        ),
    )(x2, pos_t)


def _ffn_body(eot_ref, xs_ref, w1_ref, b1_ref, w2_ref, b2_ref, out_ref):
    i = pl.program_id(0)

    @pl.when(i < eot_ref[63])
    def _():
        h = jnp.dot(xs_ref[...], w1_ref[0], preferred_element_type=jnp.float32)
        h = jnp.maximum(h + b1_ref[0], 0.0)
        out_ref[...] = jnp.dot(h, w2_ref[0],
                               preferred_element_type=jnp.float32) + b2_ref[0]


def _ffn(eot, xs, W1b, b1r, W2b, b2r):
    grid_spec = pltpu.PrefetchScalarGridSpec(
        num_scalar_prefetch=1,
        grid=(NUM_TILES,),
        in_specs=[
            pl.BlockSpec((ROW_TILE, D_MODEL), lambda i, eot: (i, 0)),
            pl.BlockSpec((1, D_MODEL, D_FF), lambda i, eot: (eot[i], 0, 0)),
            pl.BlockSpec((1, 1, D_FF), lambda i, eot: (eot[i], 0, 0)),
            pl.BlockSpec((1, D_FF, D_MODEL), lambda i, eot: (eot[i], 0, 0)),
            pl.BlockSpec((1, 1, D_MODEL), lambda i, eot: (eot[i], 0, 0)),
        ],
        out_specs=pl.BlockSpec((ROW_TILE, D_MODEL), lambda i, eot: (i, 0)),
    )
    return pl.pallas_call(
        _ffn_body,
        grid_spec=grid_spec,
        out_shape=jax.ShapeDtypeStruct((RSORT, D_MODEL), jnp.float32),
        compiler_params=pltpu.CompilerParams(
            dimension_semantics=("arbitrary",),
        ),
    )(eot, xs, W1b, b1r, W2b, b2r)


_CCH = 16


_CNC = TPW // _CCH


def _combine_body(ys_hbm, pos_hbm, sb_hbm, out_hbm,
                  y0a, y1a, y0b, y1b,
                  sbuf0, sbuf1, idx0, idx1,
                  isem, gsa, gsb, osa, osb):
    wid = lax.axis_index("s") * 2 + lax.axis_index("c")
    base0 = wid * TPW
    i0 = pltpu.async_copy(pos_hbm.at[0, pl.ds(base0, TPW)], idx0, isem)
    i1 = pltpu.async_copy(pos_hbm.at[1, pl.ds(base0, TPW)], idx1, isem)
    i2 = pltpu.async_copy(sb_hbm.at[pl.ds(base0, TPW)], sbuf0, isem)
    i3 = pltpu.async_copy(sb_hbm.at[pl.ds(TOKENS + base0, TPW)], sbuf1, isem)
    i0.wait(); i1.wait(); i2.wait(); i3.wait()
    ybufs = ((y0a, y1a), (y0b, y1b))
    gsems = (gsa, gsb)
    osems = (osa, osb)
    gaths = [None, None]
    outs = [None, None]

    def gather(ci, b):
        sl = pl.ds(ci * _CCH, _CCH)
        g0 = pltpu.async_copy(ys_hbm.at[idx0.at[sl]], ybufs[b][0], gsems[b])
        g1 = pltpu.async_copy(ys_hbm.at[idx1.at[sl]], ybufs[b][1], gsems[b])
        return (g0, g1)

    gaths[0] = gather(0, 0)
    gaths[1] = gather(1, 1)
    for ci in range(_CNC):
        b = ci % 2
        gaths[b][0].wait()
        gaths[b][1].wait()
        y0, y1 = ybufs[b]

        def row_body(j, carry, y0=y0, y1=y1, ci=ci):
            sv0 = sbuf0[pl.ds(ci * _CCH + j, 1), pl.ds(0, 16)]
            sv1 = sbuf1[pl.ds(ci * _CCH + j, 1), pl.ds(0, 16)]
            s0v = sv0.reshape((16,))
            s1v = sv1.reshape((16,))
            for cc in range(D_MODEL // 16):
                sl2 = pl.ds(cc * 16, 16)
                y0[j, sl2] = y0[j, sl2] * s0v + y1[j, sl2] * s1v
            return carry

        lax.fori_loop(0, _CCH, row_body, 0)
        outs[b] = pltpu.async_copy(
            y0, out_hbm.at[pl.ds(base0 + ci * _CCH, _CCH)], osems[b])
        if ci + 2 < _CNC:
            outs[b].wait()
            gaths[b] = gather(ci + 2, b)
    for b in range(2):
        outs[b].wait()


def _combine(ys, pos_t, sb):
    mesh = plsc.VectorSubcoreMesh(core_axis_name="c", subcore_axis_name="s")
    return pl.kernel(
        _combine_body,
        out_type=jax.ShapeDtypeStruct((TOKENS, D_MODEL), jnp.float32),
        mesh=mesh,
        scratch_types=(
            [pltpu.VMEM((_CCH, D_MODEL), jnp.float32)] * 4
            + [pltpu.VMEM((TPW, 16), jnp.float32)] * 2
            + [pltpu.VMEM((TPW,), jnp.int32)] * 2
            + [pltpu.SemaphoreType.DMA] * 5
        ),
    )(ys, pos_t, sb)


@jax.jit
def kernel(x, W1, b1, W2, b2, Wg, bg):
    B, N, D = x.shape
    x2 = x.reshape(B * N, D)
    pos01, sb, eot64 = _route(x2, Wg, bg)
    pos_t = pos01.T
    eot = eot64.reshape(64)
    xs = _dispatch(x2, pos_t)
    ys = _ffn(eot, xs, W1, b1.reshape(NUM_EXPERTS, 1, D_FF),
              W2, b2.reshape(NUM_EXPERTS, 1, D_MODEL))
    out2 = _combine(ys, pos_t, sb)
    return out2.reshape(B, N, D)
